# baseline (speedup 1.0000x reference)
import functools

import jax
import jax.numpy as jnp
from jax import lax
from jax.experimental import pallas as pl
from jax.experimental.pallas import tpu as pltpu
from jax.experimental.pallas import tpu_sc as plsc

N = 10000
E = 320000
NS = 16
CH = 80
EPT = E // NS
NCH = EPT // CH


def _sc_mesh():
    return plsc.VectorSubcoreMesh(core_axis_name="c", subcore_axis_name="s")


def _mm2_tc(x, Wl, bl, Wr, br, G, BN):
    n, din = x.shape
    hc = Wl.shape[1]
    wc = hc // G
    nb = n // BN
    wl3 = Wl.reshape(din, G, wc).transpose(1, 0, 2)
    wr3 = Wr.reshape(din, G, wc).transpose(1, 0, 2)
    bl3 = bl.reshape(G, 1, wc)
    br3 = br.reshape(G, 1, wc)

    def kern(x_ref, wl_ref, bl_ref, wr_ref, br_ref, ol_ref, or_ref):
        xv = x_ref[...]
        ol_ref[...] = jnp.dot(xv, wl_ref[0],
                              preferred_element_type=jnp.float32) + bl_ref[0]
        or_ref[...] = jnp.dot(xv, wr_ref[0],
                              preferred_element_type=jnp.float32) + br_ref[0]

    return pl.pallas_call(
        kern,
        grid=(nb, G),
        in_specs=[
            pl.BlockSpec((BN, din), lambda i, g: (i, 0)),
            pl.BlockSpec((1, din, wc), lambda i, g: (g, 0, 0)),
            pl.BlockSpec((1, 1, wc), lambda i, g: (g, 0, 0)),
            pl.BlockSpec((1, din, wc), lambda i, g: (g, 0, 0)),
            pl.BlockSpec((1, 1, wc), lambda i, g: (g, 0, 0)),
        ],
        out_specs=[
            pl.BlockSpec((BN, wc), lambda i, g, nb=nb: (g * nb + i, 0)),
            pl.BlockSpec((BN, wc), lambda i, g, nb=nb: (g * nb + i, 0)),
        ],
        out_shape=[jax.ShapeDtypeStruct((G * n, wc), jnp.float32)] * 2,
    )(x, wl3, bl3, wr3, br3)


def _emm_tc(eattr, We, G, BE):
    e_, de = eattr.shape
    hc = We.shape[1]
    wc = hc // G
    ne = e_ // BE
    w3 = We.reshape(de, G, wc).transpose(1, 0, 2)

    def kern(a_ref, w_ref, o_ref):
        o_ref[...] = jnp.dot(a_ref[...], w_ref[0],
                             preferred_element_type=jnp.float32)

    return pl.pallas_call(
        kern,
        grid=(ne, G),
        in_specs=[
            pl.BlockSpec((BE, de), lambda i, g: (i, 0)),
            pl.BlockSpec((1, de, wc), lambda i, g: (g, 0, 0)),
        ],
        out_specs=pl.BlockSpec((BE, wc), lambda i, g, ne=ne: (g * ne + i, 0)),
        out_shape=jax.ShapeDtypeStruct((G * e_, wc), jnp.float32),
    )(eattr, w3)


def _epi_tc(tbls, b, wc, h2, BN):
    n2, p = tbls[0].shape
    n = n2 // 2
    nb = n // BN
    ngr = 2 * len(tbls)
    b2 = b.reshape(1, ngr * wc)

    def kern(*refs):
        t_refs = refs[:ngr]
        b_ref = refs[ngr]
        o_ref = refs[ngr + 1]
        for g, t_ref in enumerate(t_refs):
            t = t_ref[...]
            for h in range(h2):
                cout = g * wc + 32 * h
                num = t[:, 32 * h:32 * h + 32]
                den = t[:, wc + h:wc + h + 1]
                r = num / (den + 1e-16) + b_ref[:, cout:cout + 32]
                o_ref[:, cout:cout + 32] = jnp.where(r > 0.0, r,
                                                     jnp.exp(r) - 1.0)

    in_specs = []
    args = []
    for tbl in tbls:
        in_specs.append(pl.BlockSpec((BN, p), lambda i: (i, 0)))
        in_specs.append(pl.BlockSpec((BN, p), lambda i, nb=nb: (nb + i, 0)))
        args.extend([tbl, tbl])
    in_specs.append(pl.BlockSpec((1, ngr * wc), lambda i: (0, 0)))
    args.append(b2)

    return pl.pallas_call(
        kern,
        grid=(nb,),
        in_specs=in_specs,
        out_specs=pl.BlockSpec((BN, ngr * wc), lambda i: (i, 0)),
        out_shape=jax.ShapeDtypeStruct((n, ngr * wc), jnp.float32),
    )(*args)


def _fin_tc(num, den, b):
    def kern(n_ref, d_ref, b_ref, o_ref):
        o_ref[...] = n_ref[...] / (d_ref[...] + 1e-16) + b_ref[...]

    return pl.pallas_call(
        kern,
        out_shape=jax.ShapeDtypeStruct(num.shape, jnp.float32),
    )(num, den, b)


def _sc_params():
    import dataclasses
    cp = pltpu.CompilerParams()
    fields = pltpu.CompilerParams.__dataclass_fields__
    if "needs_layout_passes" in fields:
        cp = dataclasses.replace(cp, needs_layout_passes=False)
    if "use_tc_tiling_on_sc" in fields:
        cp = dataclasses.replace(cp, use_tc_tiling_on_sc=False)
    return cp


def _gat_edges_sc(xl, xr, e, src, dst, attg, wc, h2, p, goff):
    nk = wc // 16
    nzc = N // CH

    @functools.partial(
        pl.kernel,
        out_type=jax.ShapeDtypeStruct((2 * N, p), jnp.float32),
        mesh=_sc_mesh(),
        compiler_params=_sc_params(),
        scratch_types=[
            pltpu.VMEM_SHARED((N, p), jnp.float32),
            [pltpu.VMEM((CH,), jnp.int32)] * 2,
            [pltpu.VMEM((CH,), jnp.int32)] * 2,
            [pltpu.VMEM((CH,), jnp.int32)] * 2,
            [pltpu.VMEM((CH,), jnp.int32)] * 2,
            pltpu.VMEM((CH,), jnp.int32),
            [pltpu.VMEM((CH, wc), jnp.float32)] * 2,
            [pltpu.VMEM((CH, wc), jnp.float32)] * 2,
            [pltpu.VMEM((CH, wc), jnp.float32)] * 2,
            pltpu.VMEM((CH, p), jnp.float32),
            pltpu.VMEM((wc,), jnp.float32),
            [pltpu.SemaphoreType.DMA] * 2,
            [pltpu.SemaphoreType.DMA] * 2,
            pltpu.SemaphoreType.DMA,
        ],
    )
    def kern(xl_hbm, xr_hbm, e_hbm, src_hbm, dst_hbm, att_hbm, out_hbm,
             table, src_v, dst_v, srcg_v, dstg_v, dstsc_v, xl_buf, xr_buf,
             e_buf, contrib, att_v, sem_idx, sem_dat, sem_sc):
        c = lax.axis_index("c")
        s = lax.axis_index("s")
        g = goff + c
        gn = g * N

        zf = jnp.zeros((16,), jnp.float32)

        @pl.loop(0, CH)
        def _(i):
            for k in range(p // 16):
                contrib[i, pl.ds(k * 16, 16)] = zf

        @pl.loop(0, (nzc + NS - 1) // NS)
        def _(q):
            j = s + NS * q

            @pl.when(j < nzc)
            def _():
                pltpu.sync_copy(contrib, table.at[pl.ds(j * CH, CH)])

        pltpu.sync_copy(att_hbm.at[g], att_v)
        plsc.subcore_barrier()

        att_regs = [att_v[pl.ds(k * 16, 16)] for k in range(nk)]
        lane = lax.iota(jnp.int32, 16)
        ebase = g * E

        def issue_idx(j, b):
            base = s * EPT + j * CH
            pltpu.async_copy(src_hbm.at[pl.ds(base, CH)], src_v[b],
                             sem_idx[b])
            pltpu.async_copy(dst_hbm.at[pl.ds(base, CH)], dst_v[b],
                             sem_idx[b])

        def wait_idx(b):
            base = s * EPT
            pltpu.make_async_copy(src_hbm.at[pl.ds(base, CH)], src_v[b],
                                  sem_idx[b]).wait()
            pltpu.make_async_copy(dst_hbm.at[pl.ds(base, CH)], dst_v[b],
                                  sem_idx[b]).wait()

        def issue_gathers(j, b):
            for q in range(CH // 16):
                sl = pl.ds(q * 16, 16)
                srcg_v[b][sl] = src_v[b][sl] + gn
                dstg_v[b][sl] = dst_v[b][sl] + gn
            pltpu.async_copy(xl_hbm.at[srcg_v[b]], xl_buf[b], sem_dat[b])
            pltpu.async_copy(xr_hbm.at[dstg_v[b]], xr_buf[b], sem_dat[b])
            base = s * EPT + j * CH
            pltpu.async_copy(e_hbm.at[pl.ds(ebase + base, CH)], e_buf[b],
                             sem_dat[b])

        def wait_gathers(b):
            pltpu.make_async_copy(xl_hbm.at[srcg_v[b]], xl_buf[b],
                                  sem_dat[b]).wait()
            pltpu.make_async_copy(xr_hbm.at[dstg_v[b]], xr_buf[b],
                                  sem_dat[b]).wait()
            pltpu.make_async_copy(e_hbm.at[pl.ds(s * EPT, CH)], e_buf[b],
                                  sem_dat[b]).wait()

        issue_idx(0, 0)
        issue_idx(1, 1)
        wait_idx(0)
        issue_gathers(0, 0)

        @pl.loop(0, NCH)
        def _(j):
            for b in range(2):
                @pl.when(j % 2 == b)
                def _(b=b):
                    wait_gathers(b)

                    @pl.when(j + 1 < NCH)
                    def _(b=b):
                        wait_idx(1 - b)
                        issue_gathers(j + 1, 1 - b)

                    for q in range(CH // 16):
                        sl = pl.ds(q * 16, 16)
                        dstsc_v[sl] = dst_v[b][sl]

                    @pl.when(j + 2 < NCH)
                    def _(b=b):
                        issue_idx(j + 2, b)

                    @pl.when(j > 0)
                    def _():
                        pltpu.make_async_copy(contrib, table.at[dstsc_v],
                                              sem_sc).wait()

                    zero_i = jnp.zeros((16,), jnp.int32)
                    zero_f = jnp.zeros((16,), jnp.float32)

                    @pl.loop(0, CH // 16)
                    def _(j5, b=b):
                        row = 16 * j5 + lane
                        eas = []
                        for h in range(h2):
                            def body_a(q, acc, h=h):
                                for j4 in range(4):
                                    colv = zero_i + (32 * h + 4 * q + j4)
                                    a_s = plsc.load_gather(att_v, [colv])
                                    xlt = plsc.load_gather(xl_buf[b],
                                                           [row, colv])
                                    xrt = plsc.load_gather(xr_buf[b],
                                                           [row, colv])
                                    et = plsc.load_gather(e_buf[b],
                                                          [row, colv])
                                    m = xlt + xrt + et
                                    m = jnp.where(m >= 0.0, m, m * 0.2)
                                    acc = acc + m * a_s
                                return acc
                            alpha = lax.fori_loop(0, 8, body_a, zero_f)
                            eas.append(jnp.exp(jnp.minimum(alpha, 60.0)))
                        for h in range(h2):
                            ea = eas[h]

                            def body_c(q, carry, h=h, ea=ea):
                                for j4 in range(4):
                                    colv = zero_i + (32 * h + 4 * q + j4)
                                    xlt = plsc.load_gather(xl_buf[b],
                                                           [row, colv])
                                    plsc.store_scatter(contrib, [row, colv],
                                                       xlt * ea)
                                return carry
                            lax.fori_loop(0, 8, body_c, jnp.int32(0))
                            plsc.store_scatter(
                                contrib,
                                [row, jnp.full((16,), wc + h, jnp.int32)],
                                ea)

                    pltpu.async_copy(contrib, table.at[dstsc_v], sem_sc,
                                     add=True)

        pltpu.make_async_copy(contrib, table.at[dstsc_v], sem_sc).wait()
        plsc.subcore_barrier()

        @pl.loop(0, (nzc + NS - 1) // NS)
        def _(q):
            j = s + NS * q

            @pl.when(j < nzc)
            def _():
                pltpu.sync_copy(table.at[pl.ds(j * CH, CH)],
                                out_hbm.at[pl.ds(c * N + j * CH, CH)])

    return kern(xl, xr, e, src, dst, attg)


def _gat_edges_sc3(xl, xr, e, src, dst, attp):
    nh = N // 2
    zc = 40
    nzc = nh // zc
    cpr = 312

    @functools.partial(
        pl.kernel,
        out_type=[jax.ShapeDtypeStruct((N,), jnp.float32)] * 2,
        mesh=_sc_mesh(),
        compiler_params=_sc_params(),
        scratch_types=[
            pltpu.VMEM_SHARED((nh,), jnp.float32),
            pltpu.VMEM_SHARED((nh,), jnp.float32),
            pltpu.VMEM((N,), jnp.float32),
            pltpu.VMEM((N,), jnp.float32),
            pltpu.VMEM((CH,), jnp.int32),
            pltpu.VMEM((CH,), jnp.int32),
            pltpu.VMEM((CH,), jnp.int32),
            pltpu.VMEM((CH,), jnp.float32),
            pltpu.VMEM((CH,), jnp.float32),
            pltpu.VMEM((CH,), jnp.float32),
            pltpu.VMEM((16,), jnp.float32),
            pltpu.SemaphoreType.DMA,
            pltpu.SemaphoreType.DMA,
            pltpu.SemaphoreType.DMA,
        ],
    )
    def kern(xl_hbm, xr_hbm, e_hbm, src_hbm, dst_hbm, att_hbm,
             num_hbm, den_hbm,
             num_t, den_t, xl_v, xr_v, src_v, dst_v, idx_v, e_v, cn_v, cd_v,
             att_v, sem0, sem1, sem2):
        c = lax.axis_index("c")
        s = lax.axis_index("s")
        lo = c * nh

        zf = jnp.zeros((16,), jnp.float32)
        @pl.loop(0, CH // 16)
        def _(q):
            cn_v[pl.ds(q * 16, 16)] = zf

        @pl.loop(0, 8)
        def _(q):
            j = s * 8 + q

            @pl.when(j < nzc)
            def _():
                pltpu.sync_copy(cn_v.at[pl.ds(0, zc)],
                                num_t.at[pl.ds(j * zc, zc)])
                pltpu.sync_copy(cn_v.at[pl.ds(0, zc)],
                                den_t.at[pl.ds(j * zc, zc)])

        pltpu.sync_copy(xl_hbm, xl_v)
        pltpu.sync_copy(xr_hbm, xr_v)
        pltpu.sync_copy(att_hbm, att_v)
        plsc.subcore_barrier()
        att_s = att_v[pl.ds(0, 16)][0]

        @pl.loop(0, NCH)
        def _(j):
            base = s * EPT + j * CH
            cp0 = pltpu.async_copy(src_hbm.at[pl.ds(base, CH)], src_v, sem0)
            cp1 = pltpu.async_copy(dst_hbm.at[pl.ds(base, CH)], dst_v, sem1)
            cp2 = pltpu.async_copy(e_hbm.at[pl.ds(base, CH)], e_v, sem2)
            cp0.wait()
            cp1.wait()
            cp2.wait()
            for q in range(CH // 16):
                sl = pl.ds(q * 16, 16)
                sv = src_v[sl]
                dv = dst_v[sl]
                xls = plsc.load_gather(xl_v, [sv])
                xrd = plsc.load_gather(xr_v, [dv])
                m = xls + xrd + e_v[sl]
                m = jnp.where(m >= 0.0, m, m * 0.2)
                alpha = jnp.minimum(m * att_s, 60.0)
                ea = jnp.exp(alpha)
                own = (dv >= lo) & (dv < lo + nh)
                cn_v[sl] = jnp.where(own, ea * xls, 0.0)
                cd_v[sl] = jnp.where(own, ea, 0.0)
                idx_v[sl] = jnp.where(own, dv - lo, 0)
            pltpu.sync_copy(cn_v, num_t.at[idx_v], add=True)
            pltpu.sync_copy(cd_v, den_t.at[idx_v], add=True)

        plsc.subcore_barrier()

        pltpu.sync_copy(num_t.at[pl.ds(s * cpr, cpr)],
                        num_hbm.at[pl.ds(lo + s * cpr, cpr)])
        pltpu.sync_copy(den_t.at[pl.ds(s * cpr, cpr)],
                        den_hbm.at[pl.ds(lo + s * cpr, cpr)])

        @pl.when(s == NS - 1)
        def _():
            pltpu.sync_copy(num_t.at[pl.ds(NS * cpr, nh - NS * cpr)],
                            num_hbm.at[pl.ds(lo + NS * cpr, nh - NS * cpr)])
            pltpu.sync_copy(den_t.at[pl.ds(NS * cpr, nh - NS * cpr)],
                            den_hbm.at[pl.ds(lo + NS * cpr, nh - NS * cpr)])

    return kern(xl, xr, e, src, dst, attp)


def kernel(x, edge_index, edge_attr,
           Wl1, bl1, Wr1, br1, We1, att1, b1,
           Wl2, bl2, Wr2, br2, We2, att2, b2,
           Wl3, bl3, Wr3, br3, We3, att3, b3):
    src = edge_index[0]
    dst = edge_index[1]

    xl1, xr1 = _mm2_tc(x, Wl1, bl1, Wr1, br1, G=4, BN=400)
    e1 = _emm_tc(edge_attr, We1, G=4, BE=2000)
    att1g = att1.reshape(4, 96)
    t1a = _gat_edges_sc(xl1, xr1, e1, src, dst, att1g,
                        wc=96, h2=3, p=112, goff=0)
    t1b = _gat_edges_sc(xl1, xr1, e1, src, dst, att1g,
                        wc=96, h2=3, p=112, goff=2)
    h1 = _epi_tc([t1a, t1b], b1, wc=96, h2=3, BN=400)

    xl2, xr2 = _mm2_tc(h1, Wl2, bl2, Wr2, br2, G=2, BN=400)
    e2 = _emm_tc(edge_attr, We2, G=2, BE=2000)
    t2 = _gat_edges_sc(xl2, xr2, e2, src, dst, att2.reshape(2, 96),
                       wc=96, h2=3, p=112, goff=0)
    h2 = _epi_tc([t2], b2, wc=96, h2=3, BN=400)

    xl3, xr3 = _mm2_tc(h2, Wl3, bl3, Wr3, br3, G=1, BN=2000)
    e3 = _emm_tc(edge_attr, We3, G=1, BE=2000)
    att3p = jnp.pad(att3.reshape(1), (0, 15))
    num3, den3 = _gat_edges_sc3(xl3.reshape(-1), xr3.reshape(-1),
                                e3.reshape(-1), src, dst, att3p)
    out = _fin_tc(num3.reshape(-1, 1), den3.reshape(-1, 1), b3.reshape(1, 1))
    return out


# speedup vs baseline: 1.7651x; 1.7651x over previous
import functools

import jax
import jax.numpy as jnp
from jax import lax
from jax.experimental import pallas as pl
from jax.experimental.pallas import tpu as pltpu
from jax.experimental.pallas import tpu_sc as plsc

N = 10000
E = 320000
NS = 16
CH = 80
EPT = E // NS
NCH = EPT // CH


def _sc_mesh():
    return plsc.VectorSubcoreMesh(core_axis_name="c", subcore_axis_name="s")


def _mm2_tc(x, Wl, bl, Wr, br, G, BN):
    n, din = x.shape
    hc = Wl.shape[1]
    wc = hc // G
    nb = n // BN
    wl3 = Wl.reshape(din, G, wc).transpose(1, 0, 2)
    wr3 = Wr.reshape(din, G, wc).transpose(1, 0, 2)
    bl3 = bl.reshape(G, 1, wc)
    br3 = br.reshape(G, 1, wc)

    def kern(x_ref, wl_ref, bl_ref, wr_ref, br_ref, ol_ref, or_ref):
        xv = x_ref[...]
        ol_ref[...] = jnp.dot(xv, wl_ref[0],
                              preferred_element_type=jnp.float32) + bl_ref[0]
        or_ref[...] = jnp.dot(xv, wr_ref[0],
                              preferred_element_type=jnp.float32) + br_ref[0]

    return pl.pallas_call(
        kern,
        grid=(nb, G),
        in_specs=[
            pl.BlockSpec((BN, din), lambda i, g: (i, 0)),
            pl.BlockSpec((1, din, wc), lambda i, g: (g, 0, 0)),
            pl.BlockSpec((1, 1, wc), lambda i, g: (g, 0, 0)),
            pl.BlockSpec((1, din, wc), lambda i, g: (g, 0, 0)),
            pl.BlockSpec((1, 1, wc), lambda i, g: (g, 0, 0)),
        ],
        out_specs=[
            pl.BlockSpec((BN, wc), lambda i, g, nb=nb: (g * nb + i, 0)),
            pl.BlockSpec((BN, wc), lambda i, g, nb=nb: (g * nb + i, 0)),
        ],
        out_shape=[jax.ShapeDtypeStruct((G * n, wc), jnp.float32)] * 2,
    )(x, wl3, bl3, wr3, br3)


def _emm_tc(eattr, We, G, BE):
    e_, de = eattr.shape
    hc = We.shape[1]
    wc = hc // G
    ne = e_ // BE
    w3 = We.reshape(de, G, wc).transpose(1, 0, 2)

    def kern(a_ref, w_ref, o_ref):
        o_ref[...] = jnp.dot(a_ref[...], w_ref[0],
                             preferred_element_type=jnp.float32)

    return pl.pallas_call(
        kern,
        grid=(ne, G),
        in_specs=[
            pl.BlockSpec((BE, de), lambda i, g: (i, 0)),
            pl.BlockSpec((1, de, wc), lambda i, g: (g, 0, 0)),
        ],
        out_specs=pl.BlockSpec((BE, wc), lambda i, g, ne=ne: (g * ne + i, 0)),
        out_shape=jax.ShapeDtypeStruct((G * e_, wc), jnp.float32),
    )(eattr, w3)


def _epi_tc(tbls, b, wc, h2, BN):
    n2, p = tbls[0].shape
    n = n2 // 2
    nb = n // BN
    ngr = 2 * len(tbls)
    b2 = b.reshape(1, ngr * wc)

    def kern(*refs):
        t_refs = refs[:ngr]
        b_ref = refs[ngr]
        o_ref = refs[ngr + 1]
        for g, t_ref in enumerate(t_refs):
            t = t_ref[...]
            for h in range(h2):
                cout = g * wc + 32 * h
                num = t[:, 32 * h:32 * h + 32]
                den = t[:, wc + h:wc + h + 1]
                r = num / (den + 1e-16) + b_ref[:, cout:cout + 32]
                o_ref[:, cout:cout + 32] = jnp.where(r > 0.0, r,
                                                     jnp.exp(r) - 1.0)

    in_specs = []
    args = []
    for tbl in tbls:
        in_specs.append(pl.BlockSpec((BN, p), lambda i: (i, 0)))
        in_specs.append(pl.BlockSpec((BN, p), lambda i, nb=nb: (nb + i, 0)))
        args.extend([tbl, tbl])
    in_specs.append(pl.BlockSpec((1, ngr * wc), lambda i: (0, 0)))
    args.append(b2)

    return pl.pallas_call(
        kern,
        grid=(nb,),
        in_specs=in_specs,
        out_specs=pl.BlockSpec((BN, ngr * wc), lambda i: (i, 0)),
        out_shape=jax.ShapeDtypeStruct((n, ngr * wc), jnp.float32),
    )(*args)


def _fin_tc(num, den, b):
    def kern(n_ref, d_ref, b_ref, o_ref):
        o_ref[...] = n_ref[...] / (d_ref[...] + 1e-16) + b_ref[...]

    return pl.pallas_call(
        kern,
        out_shape=jax.ShapeDtypeStruct(num.shape, jnp.float32),
    )(num, den, b)


def _sc_params():
    import dataclasses
    cp = pltpu.CompilerParams()
    fields = pltpu.CompilerParams.__dataclass_fields__
    if "needs_layout_passes" in fields:
        cp = dataclasses.replace(cp, needs_layout_passes=False)
    if "use_tc_tiling_on_sc" in fields:
        cp = dataclasses.replace(cp, use_tc_tiling_on_sc=False)
    return cp


def _gat_edges_sc(xl, xr, e, src, dst, attg, wc, h2, p, goff):
    nk = wc // 16
    nzc = N // CH

    @functools.partial(
        pl.kernel,
        out_type=jax.ShapeDtypeStruct((2 * N, p), jnp.float32),
        mesh=_sc_mesh(),
        compiler_params=_sc_params(),
        scratch_types=[
            pltpu.VMEM_SHARED((N, p), jnp.float32),
            [pltpu.VMEM((CH,), jnp.int32)] * 2,
            [pltpu.VMEM((CH,), jnp.int32)] * 2,
            [pltpu.VMEM((CH,), jnp.int32)] * 2,
            [pltpu.VMEM((CH,), jnp.int32)] * 2,
            pltpu.VMEM((CH,), jnp.int32),
            [pltpu.VMEM((CH, wc), jnp.float32)] * 2,
            [pltpu.VMEM((CH, wc), jnp.float32)] * 2,
            [pltpu.VMEM((CH, wc), jnp.float32)] * 2,
            pltpu.VMEM((CH, p), jnp.float32),
            pltpu.VMEM((wc,), jnp.float32),
            [pltpu.SemaphoreType.DMA] * 2,
            [pltpu.SemaphoreType.DMA] * 2,
            pltpu.SemaphoreType.DMA,
        ],
    )
    def kern(xl_hbm, xr_hbm, e_hbm, src_hbm, dst_hbm, att_hbm, out_hbm,
             table, src_v, dst_v, srcg_v, dstg_v, dstsc_v, xl_buf, xr_buf,
             e_buf, contrib, att_v, sem_idx, sem_dat, sem_sc):
        c = lax.axis_index("c")
        s = lax.axis_index("s")
        g = goff + c
        gn = g * N

        zf = jnp.zeros((16,), jnp.float32)

        @pl.loop(0, CH)
        def _(i):
            for k in range(p // 16):
                contrib[i, pl.ds(k * 16, 16)] = zf

        @pl.loop(0, (nzc + NS - 1) // NS)
        def _(q):
            j = s + NS * q

            @pl.when(j < nzc)
            def _():
                pltpu.sync_copy(contrib, table.at[pl.ds(j * CH, CH)])

        pltpu.sync_copy(att_hbm.at[g], att_v)
        plsc.subcore_barrier()

        att_regs = [att_v[pl.ds(k * 16, 16)] for k in range(nk)]
        lane = lax.iota(jnp.int32, 16)
        ebase = g * E

        def issue_idx(j, b):
            base = s * EPT + j * CH
            pltpu.async_copy(src_hbm.at[pl.ds(base, CH)], src_v[b],
                             sem_idx[b])
            pltpu.async_copy(dst_hbm.at[pl.ds(base, CH)], dst_v[b],
                             sem_idx[b])

        def wait_idx(b):
            base = s * EPT
            pltpu.make_async_copy(src_hbm.at[pl.ds(base, CH)], src_v[b],
                                  sem_idx[b]).wait()
            pltpu.make_async_copy(dst_hbm.at[pl.ds(base, CH)], dst_v[b],
                                  sem_idx[b]).wait()

        def issue_gathers(j, b):
            for q in range(CH // 16):
                sl = pl.ds(q * 16, 16)
                srcg_v[b][sl] = src_v[b][sl] + gn
                dstg_v[b][sl] = dst_v[b][sl] + gn
            pltpu.async_copy(xl_hbm.at[srcg_v[b]], xl_buf[b], sem_dat[b])
            pltpu.async_copy(xr_hbm.at[dstg_v[b]], xr_buf[b], sem_dat[b])
            base = s * EPT + j * CH
            pltpu.async_copy(e_hbm.at[pl.ds(ebase + base, CH)], e_buf[b],
                             sem_dat[b])

        def wait_gathers(b):
            pltpu.make_async_copy(xl_hbm.at[srcg_v[b]], xl_buf[b],
                                  sem_dat[b]).wait()
            pltpu.make_async_copy(xr_hbm.at[dstg_v[b]], xr_buf[b],
                                  sem_dat[b]).wait()
            pltpu.make_async_copy(e_hbm.at[pl.ds(s * EPT, CH)], e_buf[b],
                                  sem_dat[b]).wait()

        issue_idx(0, 0)
        issue_idx(1, 1)
        wait_idx(0)
        issue_gathers(0, 0)

        @pl.loop(0, NCH)
        def _(j):
            for b in range(2):
                @pl.when(j % 2 == b)
                def _(b=b):
                    wait_gathers(b)

                    @pl.when(j + 1 < NCH)
                    def _(b=b):
                        wait_idx(1 - b)
                        issue_gathers(j + 1, 1 - b)

                    for q in range(CH // 16):
                        sl = pl.ds(q * 16, 16)
                        dstsc_v[sl] = dst_v[b][sl]

                    @pl.when(j + 2 < NCH)
                    def _(b=b):
                        issue_idx(j + 2, b)

                    @pl.when(j > 0)
                    def _():
                        pltpu.make_async_copy(contrib, table.at[dstsc_v],
                                              sem_sc).wait()

                    @pl.loop(0, CH // 4)
                    def _(ii, b=b):
                        for i4 in range(4):
                            i = 4 * ii + i4
                            xlv = [xl_buf[b][i, pl.ds(k * 16, 16)]
                                   for k in range(nk)]
                            tail = jnp.zeros((16,), jnp.float32)
                            for h in range(h2):
                                acc = None
                                for k2 in range(2):
                                    k = 2 * h + k2
                                    sl = pl.ds(k * 16, 16)
                                    m = (xlv[k] + xr_buf[b][i, sl]
                                         + e_buf[b][i, sl])
                                    m = jnp.where(m >= 0.0, m, m * 0.2)
                                    pr = m * att_regs[k]
                                    acc = pr if acc is None else acc + pr
                                alpha = jnp.minimum(jnp.sum(acc), 60.0)
                                ea = jnp.exp(jnp.full((16,), alpha,
                                                      jnp.float32))
                                contrib[i, pl.ds(32 * h, 16)] = (
                                    xlv[2 * h] * ea)
                                contrib[i, pl.ds(32 * h + 16, 16)] = (
                                    xlv[2 * h + 1] * ea)
                                tail = jnp.where(lane == h, ea, tail)
                            contrib[i, pl.ds(wc, 16)] = tail

                    pltpu.async_copy(contrib, table.at[dstsc_v], sem_sc,
                                     add=True)

        pltpu.make_async_copy(contrib, table.at[dstsc_v], sem_sc).wait()
        plsc.subcore_barrier()

        @pl.loop(0, (nzc + NS - 1) // NS)
        def _(q):
            j = s + NS * q

            @pl.when(j < nzc)
            def _():
                pltpu.sync_copy(table.at[pl.ds(j * CH, CH)],
                                out_hbm.at[pl.ds(c * N + j * CH, CH)])

    return kern(xl, xr, e, src, dst, attg)


def _gat_edges_sc3(xl, xr, e, src, dst, attp):
    nh = N // 2
    zc = 40
    nzc = nh // zc
    cpr = 312

    @functools.partial(
        pl.kernel,
        out_type=[jax.ShapeDtypeStruct((N,), jnp.float32)] * 2,
        mesh=_sc_mesh(),
        compiler_params=_sc_params(),
        scratch_types=[
            pltpu.VMEM_SHARED((nh,), jnp.float32),
            pltpu.VMEM_SHARED((nh,), jnp.float32),
            pltpu.VMEM((N,), jnp.float32),
            pltpu.VMEM((N,), jnp.float32),
            pltpu.VMEM((CH,), jnp.int32),
            pltpu.VMEM((CH,), jnp.int32),
            pltpu.VMEM((CH,), jnp.int32),
            pltpu.VMEM((CH,), jnp.float32),
            pltpu.VMEM((CH,), jnp.float32),
            pltpu.VMEM((CH,), jnp.float32),
            pltpu.VMEM((16,), jnp.float32),
            pltpu.SemaphoreType.DMA,
            pltpu.SemaphoreType.DMA,
            pltpu.SemaphoreType.DMA,
        ],
    )
    def kern(xl_hbm, xr_hbm, e_hbm, src_hbm, dst_hbm, att_hbm,
             num_hbm, den_hbm,
             num_t, den_t, xl_v, xr_v, src_v, dst_v, idx_v, e_v, cn_v, cd_v,
             att_v, sem0, sem1, sem2):
        c = lax.axis_index("c")
        s = lax.axis_index("s")
        lo = c * nh

        zf = jnp.zeros((16,), jnp.float32)
        @pl.loop(0, CH // 16)
        def _(q):
            cn_v[pl.ds(q * 16, 16)] = zf

        @pl.loop(0, 8)
        def _(q):
            j = s * 8 + q

            @pl.when(j < nzc)
            def _():
                pltpu.sync_copy(cn_v.at[pl.ds(0, zc)],
                                num_t.at[pl.ds(j * zc, zc)])
                pltpu.sync_copy(cn_v.at[pl.ds(0, zc)],
                                den_t.at[pl.ds(j * zc, zc)])

        pltpu.sync_copy(xl_hbm, xl_v)
        pltpu.sync_copy(xr_hbm, xr_v)
        pltpu.sync_copy(att_hbm, att_v)
        plsc.subcore_barrier()
        att_s = att_v[pl.ds(0, 16)][0]

        @pl.loop(0, NCH)
        def _(j):
            base = s * EPT + j * CH
            cp0 = pltpu.async_copy(src_hbm.at[pl.ds(base, CH)], src_v, sem0)
            cp1 = pltpu.async_copy(dst_hbm.at[pl.ds(base, CH)], dst_v, sem1)
            cp2 = pltpu.async_copy(e_hbm.at[pl.ds(base, CH)], e_v, sem2)
            cp0.wait()
            cp1.wait()
            cp2.wait()
            for q in range(CH // 16):
                sl = pl.ds(q * 16, 16)
                sv = src_v[sl]
                dv = dst_v[sl]
                xls = plsc.load_gather(xl_v, [sv])
                xrd = plsc.load_gather(xr_v, [dv])
                m = xls + xrd + e_v[sl]
                m = jnp.where(m >= 0.0, m, m * 0.2)
                alpha = jnp.minimum(m * att_s, 60.0)
                ea = jnp.exp(alpha)
                own = (dv >= lo) & (dv < lo + nh)
                cn_v[sl] = jnp.where(own, ea * xls, 0.0)
                cd_v[sl] = jnp.where(own, ea, 0.0)
                idx_v[sl] = jnp.where(own, dv - lo, 0)
            pltpu.sync_copy(cn_v, num_t.at[idx_v], add=True)
            pltpu.sync_copy(cd_v, den_t.at[idx_v], add=True)

        plsc.subcore_barrier()

        pltpu.sync_copy(num_t.at[pl.ds(s * cpr, cpr)],
                        num_hbm.at[pl.ds(lo + s * cpr, cpr)])
        pltpu.sync_copy(den_t.at[pl.ds(s * cpr, cpr)],
                        den_hbm.at[pl.ds(lo + s * cpr, cpr)])

        @pl.when(s == NS - 1)
        def _():
            pltpu.sync_copy(num_t.at[pl.ds(NS * cpr, nh - NS * cpr)],
                            num_hbm.at[pl.ds(lo + NS * cpr, nh - NS * cpr)])
            pltpu.sync_copy(den_t.at[pl.ds(NS * cpr, nh - NS * cpr)],
                            den_hbm.at[pl.ds(lo + NS * cpr, nh - NS * cpr)])

    return kern(xl, xr, e, src, dst, attp)


def kernel(x, edge_index, edge_attr,
           Wl1, bl1, Wr1, br1, We1, att1, b1,
           Wl2, bl2, Wr2, br2, We2, att2, b2,
           Wl3, bl3, Wr3, br3, We3, att3, b3):
    src = edge_index[0]
    dst = edge_index[1]

    xl1, xr1 = _mm2_tc(x, Wl1, bl1, Wr1, br1, G=4, BN=400)
    e1 = _emm_tc(edge_attr, We1, G=4, BE=2000)
    att1g = att1.reshape(4, 96)
    t1a = _gat_edges_sc(xl1, xr1, e1, src, dst, att1g,
                        wc=96, h2=3, p=112, goff=0)
    t1b = _gat_edges_sc(xl1, xr1, e1, src, dst, att1g,
                        wc=96, h2=3, p=112, goff=2)
    h1 = _epi_tc([t1a, t1b], b1, wc=96, h2=3, BN=400)

    xl2, xr2 = _mm2_tc(h1, Wl2, bl2, Wr2, br2, G=2, BN=400)
    e2 = _emm_tc(edge_attr, We2, G=2, BE=2000)
    t2 = _gat_edges_sc(xl2, xr2, e2, src, dst, att2.reshape(2, 96),
                       wc=96, h2=3, p=112, goff=0)
    h2 = _epi_tc([t2], b2, wc=96, h2=3, BN=400)

    xl3, xr3 = _mm2_tc(h2, Wl3, bl3, Wr3, br3, G=1, BN=2000)
    e3 = _emm_tc(edge_attr, We3, G=1, BE=2000)
    att3p = jnp.pad(att3.reshape(1), (0, 15))
    num3, den3 = _gat_edges_sc3(xl3.reshape(-1), xr3.reshape(-1),
                                e3.reshape(-1), src, dst, att3p)
    out = _fin_tc(num3.reshape(-1, 1), den3.reshape(-1, 1), b3.reshape(1, 1))
    return out
